# baseline (speedup 1.0000x reference)
import functools

import jax
import jax.numpy as jnp
from jax import lax
from jax.experimental import pallas as pl
from jax.experimental.pallas import tpu as pltpu
from jax.experimental.pallas import tpu_sc as plsc

_CP = pltpu.CompilerParams(use_tc_tiling_on_sc=True, needs_layout_passes=False)
_NS = 4


def _make_repack(vocab: int, emb: int):
    info = plsc.get_sparse_core_info()
    nc, ns = info.num_cores, info.num_subcores
    nw = nc * ns
    CH = 2 * emb
    n_reg = vocab // CH
    tail = vocab - n_reg * CH
    per_w_max = (n_reg + nw - 1) // nw
    n_iter = (per_w_max + _NS - 1) // _NS

    mesh = plsc.VectorSubcoreMesh(core_axis_name="c", subcore_axis_name="s")

    @functools.partial(
        pl.kernel,
        mesh=mesh,
        out_type=jax.ShapeDtypeStruct((vocab // 2, 2 * emb), jnp.float32),
        scratch_types=(
            [pltpu.VMEM((emb, CH), jnp.float32) for _ in range(_NS)]
            + [pltpu.VMEM((emb, CH), jnp.float32) for _ in range(_NS)]
            + [pltpu.VMEM((emb, emb), jnp.float32),
               pltpu.VMEM((emb // 2, CH), jnp.float32)]
            + [pltpu.SemaphoreType.DMA for _ in range(2 * _NS)]
        ),
        compiler_params=_CP,
    )
    def repack_kernel(wt_hbm, w2_hbm, *scr):
        blk = scr[0:_NS]
        tblk = scr[_NS:2 * _NS]
        tb, ttb = scr[2 * _NS], scr[2 * _NS + 1]
        lsem = scr[2 * _NS + 2:3 * _NS + 2]
        ssem = scr[3 * _NS + 2:4 * _NS + 2]
        wid = lax.axis_index("s") * nc + lax.axis_index("c")
        iota = lax.iota(jnp.int32, 16)

        def t_of(i):
            return i * nw + wid

        def valid(i):
            return t_of(i) < n_reg

        def load(i, p, start):
            cp = pltpu.make_async_copy(
                wt_hbm.at[:, pl.ds(t_of(i) * CH, CH)], blk[p], lsem[p])
            cp.start() if start else cp.wait()

        def transpose(src, dst, nu):
            for u in range(nu):
                tvs = [
                    plsc.load_gather(
                        src,
                        [iota + 16 * (k % 4),
                         jnp.full((16,), 2 * u + (1 if k >= 4 else 0),
                                  jnp.int32)])
                    for k in range(8)
                ]
                for k in range(8):
                    dst[u, pl.ds(16 * k, 16)] = tvs[k]

        def store(i, p, start):
            cp = pltpu.make_async_copy(
                tblk[p], w2_hbm.at[pl.ds(t_of(i) * emb, emb), :], ssem[p])
            cp.start() if start else cp.wait()

        for p in range(_NS):
            @pl.when(valid(p))
            def _(p=p):
                load(p, p, True)

        def body(j, carry):
            for p in range(_NS):
                i = _NS * j + p

                @pl.when(valid(i))
                def _(i=i, p=p):
                    load(i, p, False)

                @pl.when((j >= 1) & valid(i - _NS))
                def _(i=i, p=p):
                    store(i - _NS, p, False)

                @pl.when(valid(i))
                def _(i=i, p=p):
                    transpose(blk[p], tblk[p], emb)
                    store(i, p, True)

                @pl.when(valid(i + _NS))
                def _(i=i, p=p):
                    load(i + _NS, p, True)
            return carry

        lax.fori_loop(0, n_iter, body, 0)

        for p in range(_NS):
            i = _NS * (n_iter - 1) + p

            @pl.when(valid(i))
            def _(i=i, p=p):
                store(i, p, False)

        if tail:
            @pl.when(wid == 0)
            def _():
                pltpu.sync_copy(wt_hbm.at[:, pl.ds(n_reg * CH, tail)], tb)
                transpose(tb, ttb, tail // 2)
                pltpu.sync_copy(
                    ttb, w2_hbm.at[pl.ds(n_reg * emb, tail // 2), :])

    return repack_kernel


def _make_lookup(vocab: int, emb: int, b_dim: int, l_dim: int):
    info = plsc.get_sparse_core_info()
    nc, ns = info.num_cores, info.num_subcores
    nw = nc * ns
    CH = 128
    n_bc = b_dim // CH
    n_chunks = l_dim * n_bc
    per_w = n_chunks // nw
    assert n_chunks % (nw * _NS) == 0
    n_iter = per_w // _NS

    mesh = plsc.VectorSubcoreMesh(core_axis_name="c", subcore_axis_name="s")

    @functools.partial(
        pl.kernel,
        mesh=mesh,
        out_type=jax.ShapeDtypeStruct((l_dim, emb, b_dim), jnp.float32),
        scratch_types=(
            [pltpu.VMEM((CH,), jnp.int32) for _ in range(3 * _NS)]
            + [pltpu.VMEM((CH, 2 * emb), jnp.float32) for _ in range(_NS)]
            + [pltpu.VMEM((emb, CH), jnp.float32) for _ in range(_NS)]
            + [pltpu.SemaphoreType.DMA for _ in range(3 * _NS)]
        ),
        compiler_params=_CP,
    )
    def lookup_kernel(mt_hbm, w2_hbm, out_hbm, *scr):
        idxraw = scr[0:_NS]
        gidx = scr[_NS:2 * _NS]
        par = scr[2 * _NS:3 * _NS]
        rows = scr[3 * _NS:4 * _NS]
        trows = scr[4 * _NS:5 * _NS]
        isem = scr[5 * _NS:6 * _NS]
        gsem = scr[6 * _NS:7 * _NS]
        osem = scr[7 * _NS:8 * _NS]
        wid = lax.axis_index("s") * nc + lax.axis_index("c")
        base = wid * per_w
        iota = lax.iota(jnp.int32, 16)

        def lc(t):
            return lax.div(t, n_bc), lax.rem(t, n_bc) * CH

        def idx_load(t, p, start):
            l, c = lc(base + t)
            cp = pltpu.make_async_copy(
                mt_hbm.at[pl.ds(l * b_dim + c, CH)], idxraw[p], isem[p])
            cp.start() if start else cp.wait()

        def process(p):
            for k in range(0, CH, 16):
                v = idxraw[p][pl.ds(k, 16)]
                gidx[p][pl.ds(k, 16)] = lax.shift_right_logical(v, 1)
                par[p][pl.ds(k, 16)] = lax.shift_left(
                    jnp.bitwise_and(v, 1), 6)

        def gather(p, start):
            cp = pltpu.make_async_copy(w2_hbm.at[gidx[p]], rows[p], gsem[p])
            cp.start() if start else cp.wait()

        def transpose(p):
            for k in range(0, CH, 16):
                rowk = iota + k
                pk = par[p][pl.ds(k, 16)]
                for e0 in range(0, emb, 8):
                    tvs = [
                        plsc.load_gather(rows[p], [rowk, pk + (e0 + d)])
                        for d in range(8)
                    ]
                    for d in range(8):
                        trows[p][e0 + d, pl.ds(k, 16)] = tvs[d]

        def store(t, p, start):
            l, c = lc(base + t)
            cp = pltpu.make_async_copy(
                trows[p], out_hbm.at[l, :, pl.ds(c, CH)], osem[p])
            cp.start() if start else cp.wait()

        for p in range(_NS):
            idx_load(p, p, True)
        for p in range(_NS - 1):
            idx_load(p, p, False)
            process(p)
            gather(p, True)

        def body(j, carry):
            for p in range(_NS):
                t = _NS * j + p
                q = (p + _NS - 1) % _NS

                gather(p, False)

                @pl.when(j >= 1)
                def _(p=p):
                    store(_NS * j + p - _NS, p, False)

                transpose(p)
                store(t, p, True)

                @pl.when(t + _NS - 1 < per_w)
                def _(t=t, q=q):
                    idx_load(t + _NS - 1, q, False)
                    process(q)
                    gather(q, True)

                @pl.when(t + _NS < per_w)
                def _(t=t, p=p):
                    idx_load(t + _NS, p, True)
            return carry

        lax.fori_loop(0, n_iter, body, 0)

        for p in range(_NS):
            store(per_w - _NS + p, p, False)

    return lookup_kernel


def kernel(mask, weight):
    b, l = mask.shape
    vocab, emb = weight.shape
    mtf = mask.T.reshape(-1)
    wt = weight.T
    w2 = _make_repack(vocab, emb)(wt)
    out = _make_lookup(vocab, emb, b, l)(mtf, w2)
    return out.transpose(2, 0, 1)


# speedup vs baseline: 1.9229x; 1.9229x over previous
import functools

import jax
import jax.numpy as jnp
from jax import lax
from jax.experimental import pallas as pl
from jax.experimental.pallas import tpu as pltpu
from jax.experimental.pallas import tpu_sc as plsc

_CP = pltpu.CompilerParams(use_tc_tiling_on_sc=True, needs_layout_passes=False)
_NS = 4


def _make_repack(vocab: int, emb: int):
    info = plsc.get_sparse_core_info()
    nc, ns = info.num_cores, info.num_subcores
    nw = nc * ns
    CH = 2 * emb
    n_reg = vocab // CH
    tail = vocab - n_reg * CH
    per_w_max = (n_reg + nw - 1) // nw
    n_iter = (per_w_max + _NS - 1) // _NS

    mesh = plsc.VectorSubcoreMesh(core_axis_name="c", subcore_axis_name="s")

    @functools.partial(
        pl.kernel,
        mesh=mesh,
        out_type=jax.ShapeDtypeStruct((vocab // 2, 2 * emb), jnp.float32),
        scratch_types=(
            [pltpu.VMEM((emb, CH), jnp.float32) for _ in range(_NS)]
            + [pltpu.VMEM((emb, CH), jnp.float32) for _ in range(_NS)]
            + [pltpu.VMEM((emb, emb), jnp.float32),
               pltpu.VMEM((emb // 2, CH), jnp.float32)]
            + [pltpu.SemaphoreType.DMA for _ in range(2 * _NS)]
        ),
        compiler_params=_CP,
    )
    def repack_kernel(wt_hbm, w2_hbm, *scr):
        blk = scr[0:_NS]
        tblk = scr[_NS:2 * _NS]
        tb, ttb = scr[2 * _NS], scr[2 * _NS + 1]
        lsem = scr[2 * _NS + 2:3 * _NS + 2]
        ssem = scr[3 * _NS + 2:4 * _NS + 2]
        wid = lax.axis_index("s") * nc + lax.axis_index("c")
        iota = lax.iota(jnp.int32, 16)

        def t_of(i):
            return i * nw + wid

        def valid(i):
            return t_of(i) < n_reg

        def load(i, p, start):
            cp = pltpu.make_async_copy(
                wt_hbm.at[:, pl.ds(t_of(i) * CH, CH)], blk[p], lsem[p])
            cp.start() if start else cp.wait()

        def transpose(src, dst, nu):
            ncol = 2 * nu

            def sbody(s, carry):
                rot = jnp.bitwise_and(iota + s, 15)
                for c0 in range(0, ncol, 16):
                    cvec = iota + c0
                    u_row = lax.shift_right_logical(cvec, 1)
                    j_base = lax.shift_left(jnp.bitwise_and(cvec, 1), 6)
                    for e0 in range(0, emb, 16):
                        erow = rot + e0
                        tv = plsc.load_gather(src, [erow, cvec])
                        plsc.store_scatter(dst, [u_row, j_base + erow], tv)
                return carry

            lax.fori_loop(0, 16, sbody, 0)

        def store(i, p, start):
            cp = pltpu.make_async_copy(
                tblk[p], w2_hbm.at[pl.ds(t_of(i) * emb, emb), :], ssem[p])
            cp.start() if start else cp.wait()

        for p in range(_NS):
            @pl.when(valid(p))
            def _(p=p):
                load(p, p, True)

        def body(j, carry):
            for p in range(_NS):
                i = _NS * j + p

                @pl.when(valid(i))
                def _(i=i, p=p):
                    load(i, p, False)

                @pl.when((j >= 1) & valid(i - _NS))
                def _(i=i, p=p):
                    store(i - _NS, p, False)

                @pl.when(valid(i))
                def _(i=i, p=p):
                    transpose(blk[p], tblk[p], emb)
                    store(i, p, True)

                @pl.when(valid(i + _NS))
                def _(i=i, p=p):
                    load(i + _NS, p, True)
            return carry

        lax.fori_loop(0, n_iter, body, 0)

        for p in range(_NS):
            i = _NS * (n_iter - 1) + p

            @pl.when(valid(i))
            def _(i=i, p=p):
                store(i, p, False)

        if tail:
            @pl.when(wid == 0)
            def _():
                pltpu.sync_copy(wt_hbm.at[:, pl.ds(n_reg * CH, tail)], tb)
                transpose(tb, ttb, tail // 2)
                pltpu.sync_copy(
                    ttb, w2_hbm.at[pl.ds(n_reg * emb, tail // 2), :])

    return repack_kernel


def _make_lookup(vocab: int, emb: int, b_dim: int, l_dim: int):
    info = plsc.get_sparse_core_info()
    nc, ns = info.num_cores, info.num_subcores
    nw = nc * ns
    CH = 128
    n_bc = b_dim // CH
    n_chunks = l_dim * n_bc
    per_w = n_chunks // nw
    assert n_chunks % (nw * _NS) == 0
    n_iter = per_w // _NS

    mesh = plsc.VectorSubcoreMesh(core_axis_name="c", subcore_axis_name="s")

    @functools.partial(
        pl.kernel,
        mesh=mesh,
        out_type=jax.ShapeDtypeStruct((l_dim, emb, b_dim), jnp.float32),
        scratch_types=(
            [pltpu.VMEM((CH,), jnp.int32) for _ in range(3 * _NS)]
            + [pltpu.VMEM((CH, 2 * emb), jnp.float32) for _ in range(_NS)]
            + [pltpu.VMEM((emb, CH), jnp.float32) for _ in range(_NS)]
            + [pltpu.SemaphoreType.DMA for _ in range(3 * _NS)]
        ),
        compiler_params=_CP,
    )
    def lookup_kernel(mt_hbm, w2_hbm, out_hbm, *scr):
        idxraw = scr[0:_NS]
        gidx = scr[_NS:2 * _NS]
        par = scr[2 * _NS:3 * _NS]
        rows = scr[3 * _NS:4 * _NS]
        trows = scr[4 * _NS:5 * _NS]
        isem = scr[5 * _NS:6 * _NS]
        gsem = scr[6 * _NS:7 * _NS]
        osem = scr[7 * _NS:8 * _NS]
        wid = lax.axis_index("s") * nc + lax.axis_index("c")
        base = wid * per_w
        iota = lax.iota(jnp.int32, 16)

        def lc(t):
            return lax.div(t, n_bc), lax.rem(t, n_bc) * CH

        def idx_load(t, p, start):
            l, c = lc(base + t)
            cp = pltpu.make_async_copy(
                mt_hbm.at[pl.ds(l * b_dim + c, CH)], idxraw[p], isem[p])
            cp.start() if start else cp.wait()

        def process(p):
            for k in range(0, CH, 16):
                v = idxraw[p][pl.ds(k, 16)]
                gidx[p][pl.ds(k, 16)] = lax.shift_right_logical(v, 1)
                par[p][pl.ds(k, 16)] = lax.shift_left(
                    jnp.bitwise_and(v, 1), 6)

        def gather(p, start):
            cp = pltpu.make_async_copy(w2_hbm.at[gidx[p]], rows[p], gsem[p])
            cp.start() if start else cp.wait()

        def transpose(p):
            def sbody(s, carry):
                rot = jnp.bitwise_and(iota + s, 15)
                for k in range(0, CH, 16):
                    rowk = iota + k
                    pk = par[p][pl.ds(k, 16)]
                    for e0 in range(0, emb, 16):
                        erow = rot + e0
                        tv = plsc.load_gather(rows[p], [rowk, pk + erow])
                        plsc.store_scatter(trows[p], [erow, rowk], tv)
                return carry

            lax.fori_loop(0, 16, sbody, 0)

        def store(t, p, start):
            l, c = lc(base + t)
            cp = pltpu.make_async_copy(
                trows[p], out_hbm.at[l, :, pl.ds(c, CH)], osem[p])
            cp.start() if start else cp.wait()

        for p in range(_NS):
            idx_load(p, p, True)
        for p in range(_NS - 1):
            idx_load(p, p, False)
            process(p)
            gather(p, True)

        def body(j, carry):
            for p in range(_NS):
                t = _NS * j + p
                q = (p + _NS - 1) % _NS

                gather(p, False)

                @pl.when(j >= 1)
                def _(p=p):
                    store(_NS * j + p - _NS, p, False)

                transpose(p)
                store(t, p, True)

                @pl.when(t + _NS - 1 < per_w)
                def _(t=t, q=q):
                    idx_load(t + _NS - 1, q, False)
                    process(q)
                    gather(q, True)

                @pl.when(t + _NS < per_w)
                def _(t=t, p=p):
                    idx_load(t + _NS, p, True)
            return carry

        lax.fori_loop(0, n_iter, body, 0)

        for p in range(_NS):
            store(per_w - _NS + p, p, False)

    return lookup_kernel


def kernel(mask, weight):
    b, l = mask.shape
    vocab, emb = weight.shape
    mtf = mask.T.reshape(-1)
    wt = weight.T
    w2 = _make_repack(vocab, emb)(wt)
    out = _make_lookup(vocab, emb, b, l)(mtf, w2)
    return out.transpose(2, 0, 1)


# speedup vs baseline: 3.8591x; 2.0069x over previous
import functools

import jax
import jax.numpy as jnp
from jax import lax
from jax.experimental import pallas as pl
from jax.experimental.pallas import tpu as pltpu
from jax.experimental.pallas import tpu_sc as plsc

_CP = pltpu.CompilerParams(use_tc_tiling_on_sc=True, needs_layout_passes=False)
_NS = 4


def _make_repack(vocab: int, emb: int):
    info = plsc.get_sparse_core_info()
    nc, ns = info.num_cores, info.num_subcores
    nw = nc * ns
    CH = 2 * emb
    n_reg = vocab // CH
    tail = vocab - n_reg * CH
    per_w_max = (n_reg + nw - 1) // nw
    n_iter = (per_w_max + _NS - 1) // _NS

    mesh = plsc.VectorSubcoreMesh(core_axis_name="c", subcore_axis_name="s")

    @functools.partial(
        pl.kernel,
        mesh=mesh,
        out_type=jax.ShapeDtypeStruct((vocab // 2, 2 * emb), jnp.float32),
        scratch_types=(
            [pltpu.VMEM((emb, CH), jnp.float32) for _ in range(_NS)]
            + [pltpu.VMEM((emb, CH), jnp.float32) for _ in range(_NS)]
            + [pltpu.VMEM((emb, emb), jnp.float32),
               pltpu.VMEM((emb // 2, CH), jnp.float32)]
            + [pltpu.SemaphoreType.DMA for _ in range(2 * _NS)]
        ),
        compiler_params=_CP,
    )
    def repack_kernel(wt_hbm, w2_hbm, *scr):
        blk = scr[0:_NS]
        tblk = scr[_NS:2 * _NS]
        tb, ttb = scr[2 * _NS], scr[2 * _NS + 1]
        lsem = scr[2 * _NS + 2:3 * _NS + 2]
        ssem = scr[3 * _NS + 2:4 * _NS + 2]
        wid = lax.axis_index("s") * nc + lax.axis_index("c")
        iota = lax.iota(jnp.int32, 16)

        def t_of(i):
            return i * nw + wid

        def valid(i):
            return t_of(i) < n_reg

        def load(i, p, start):
            cp = pltpu.make_async_copy(
                wt_hbm.at[:, pl.ds(t_of(i) * CH, CH)], blk[p], lsem[p])
            cp.start() if start else cp.wait()

        def transpose(src, dst, nu):
            ncol = 2 * nu

            def sbody(s, carry):
                rot = jnp.bitwise_and(iota + s, 15)
                work = []
                for c0 in range(0, ncol, 16):
                    cvec = iota + c0
                    u_row = lax.shift_right_logical(cvec, 1)
                    j_base = lax.shift_left(jnp.bitwise_and(cvec, 1), 6)
                    for e0 in range(0, emb, 16):
                        erow = rot + e0
                        tv = plsc.load_gather(src, [erow, cvec])
                        work.append((u_row, j_base + erow, tv))
                for u_row, jcol, tv in work:
                    plsc.store_scatter(dst, [u_row, jcol], tv)
                return carry

            lax.fori_loop(0, 16, sbody, 0)

        def store(i, p, start):
            cp = pltpu.make_async_copy(
                tblk[p], w2_hbm.at[pl.ds(t_of(i) * emb, emb), :], ssem[p])
            cp.start() if start else cp.wait()

        for p in range(_NS):
            @pl.when(valid(p))
            def _(p=p):
                load(p, p, True)

        def body(j, carry):
            for p in range(_NS):
                i = _NS * j + p

                @pl.when(valid(i))
                def _(i=i, p=p):
                    load(i, p, False)

                @pl.when((j >= 1) & valid(i - _NS))
                def _(i=i, p=p):
                    store(i - _NS, p, False)

                @pl.when(valid(i))
                def _(i=i, p=p):
                    transpose(blk[p], tblk[p], emb)
                    store(i, p, True)

                @pl.when(valid(i + _NS))
                def _(i=i, p=p):
                    load(i + _NS, p, True)
            return carry

        lax.fori_loop(0, n_iter, body, 0)

        for p in range(_NS):
            i = _NS * (n_iter - 1) + p

            @pl.when(valid(i))
            def _(i=i, p=p):
                store(i, p, False)

        if tail:
            @pl.when(wid == 0)
            def _():
                pltpu.sync_copy(wt_hbm.at[:, pl.ds(n_reg * CH, tail)], tb)
                transpose(tb, ttb, tail // 2)
                pltpu.sync_copy(
                    ttb, w2_hbm.at[pl.ds(n_reg * emb, tail // 2), :])

    return repack_kernel


def _make_lookup(vocab: int, emb: int, b_dim: int, l_dim: int):
    info = plsc.get_sparse_core_info()
    nc, ns = info.num_cores, info.num_subcores
    nw = nc * ns
    CH = 128
    n_bc = b_dim // CH
    n_chunks = l_dim * n_bc
    per_w = n_chunks // nw
    assert n_chunks % (nw * _NS) == 0
    n_iter = per_w // _NS

    mesh = plsc.VectorSubcoreMesh(core_axis_name="c", subcore_axis_name="s")

    @functools.partial(
        pl.kernel,
        mesh=mesh,
        out_type=jax.ShapeDtypeStruct((l_dim, emb, b_dim), jnp.float32),
        scratch_types=(
            [pltpu.VMEM((CH,), jnp.int32) for _ in range(3 * _NS)]
            + [pltpu.VMEM((CH, 2 * emb), jnp.float32) for _ in range(_NS)]
            + [pltpu.VMEM((emb, CH), jnp.float32) for _ in range(_NS)]
            + [pltpu.SemaphoreType.DMA for _ in range(3 * _NS)]
        ),
        compiler_params=_CP,
    )
    def lookup_kernel(mt_hbm, w2_hbm, out_hbm, *scr):
        idxraw = scr[0:_NS]
        gidx = scr[_NS:2 * _NS]
        par = scr[2 * _NS:3 * _NS]
        rows = scr[3 * _NS:4 * _NS]
        trows = scr[4 * _NS:5 * _NS]
        isem = scr[5 * _NS:6 * _NS]
        gsem = scr[6 * _NS:7 * _NS]
        osem = scr[7 * _NS:8 * _NS]
        wid = lax.axis_index("s") * nc + lax.axis_index("c")
        base = wid * per_w
        iota = lax.iota(jnp.int32, 16)

        def lc(t):
            return lax.div(t, n_bc), lax.rem(t, n_bc) * CH

        def idx_load(t, p, start):
            l, c = lc(base + t)
            cp = pltpu.make_async_copy(
                mt_hbm.at[pl.ds(l * b_dim + c, CH)], idxraw[p], isem[p])
            cp.start() if start else cp.wait()

        def process(p):
            for k in range(0, CH, 16):
                v = idxraw[p][pl.ds(k, 16)]
                gidx[p][pl.ds(k, 16)] = lax.shift_right_logical(v, 1)
                par[p][pl.ds(k, 16)] = lax.shift_left(
                    jnp.bitwise_and(v, 1), 6)

        def gather(p, start):
            cp = pltpu.make_async_copy(w2_hbm.at[gidx[p]], rows[p], gsem[p])
            cp.start() if start else cp.wait()

        def transpose(p):
            def sbody(s, carry):
                rot = jnp.bitwise_and(iota + s, 15)
                work = []
                for k in range(0, CH, 16):
                    rowk = iota + k
                    pk = par[p][pl.ds(k, 16)]
                    for e0 in range(0, emb, 16):
                        erow = rot + e0
                        tv = plsc.load_gather(rows[p], [rowk, pk + erow])
                        work.append((erow, rowk, tv))
                for erow, rowk, tv in work:
                    plsc.store_scatter(trows[p], [erow, rowk], tv)
                return carry

            lax.fori_loop(0, 16, sbody, 0)

        def store(t, p, start):
            l, c = lc(base + t)
            cp = pltpu.make_async_copy(
                trows[p], out_hbm.at[l, :, pl.ds(c, CH)], osem[p])
            cp.start() if start else cp.wait()

        for p in range(_NS):
            idx_load(p, p, True)
        for p in range(_NS - 1):
            idx_load(p, p, False)
            process(p)
            gather(p, True)

        def body(j, carry):
            for p in range(_NS):
                t = _NS * j + p
                q = (p + _NS - 1) % _NS

                gather(p, False)

                @pl.when(j >= 1)
                def _(p=p):
                    store(_NS * j + p - _NS, p, False)

                transpose(p)
                store(t, p, True)

                @pl.when(t + _NS - 1 < per_w)
                def _(t=t, q=q):
                    idx_load(t + _NS - 1, q, False)
                    process(q)
                    gather(q, True)

                @pl.when(t + _NS < per_w)
                def _(t=t, p=p):
                    idx_load(t + _NS, p, True)
            return carry

        lax.fori_loop(0, n_iter, body, 0)

        for p in range(_NS):
            store(per_w - _NS + p, p, False)

    return lookup_kernel


def kernel(mask, weight):
    b, l = mask.shape
    vocab, emb = weight.shape
    mtf = mask.T.reshape(-1)
    wt = weight.T
    w2 = _make_repack(vocab, emb)(wt)
    out = _make_lookup(vocab, emb, b, l)(mtf, w2)
    return out.transpose(2, 0, 1)


# speedup vs baseline: 3.8965x; 1.0097x over previous
import functools

import jax
import jax.numpy as jnp
from jax import lax
from jax.experimental import pallas as pl
from jax.experimental.pallas import tpu as pltpu
from jax.experimental.pallas import tpu_sc as plsc

_CP = pltpu.CompilerParams(use_tc_tiling_on_sc=True, needs_layout_passes=False)
_NS = 4


def _make_repack(vocab: int, emb: int):
    info = plsc.get_sparse_core_info()
    nc, ns = info.num_cores, info.num_subcores
    nw = nc * ns
    CH = 2 * emb
    n_reg = vocab // CH
    tail = vocab - n_reg * CH
    per_w_max = (n_reg + nw - 1) // nw
    n_iter = (per_w_max + _NS - 1) // _NS

    mesh = plsc.VectorSubcoreMesh(core_axis_name="c", subcore_axis_name="s")

    @functools.partial(
        pl.kernel,
        mesh=mesh,
        out_type=jax.ShapeDtypeStruct((vocab // 2, 2 * emb), jnp.float32),
        scratch_types=(
            [pltpu.VMEM((emb, CH), jnp.float32) for _ in range(_NS)]
            + [pltpu.VMEM((emb, CH), jnp.float32) for _ in range(_NS)]
            + [pltpu.VMEM((emb, emb), jnp.float32),
               pltpu.VMEM((emb // 2, CH), jnp.float32)]
            + [pltpu.SemaphoreType.DMA for _ in range(2 * _NS)]
        ),
        compiler_params=_CP,
    )
    def repack_kernel(wt_hbm, w2_hbm, *scr):
        blk = scr[0:_NS]
        tblk = scr[_NS:2 * _NS]
        tb, ttb = scr[2 * _NS], scr[2 * _NS + 1]
        lsem = scr[2 * _NS + 2:3 * _NS + 2]
        ssem = scr[3 * _NS + 2:4 * _NS + 2]
        wid = lax.axis_index("s") * nc + lax.axis_index("c")
        iota = lax.iota(jnp.int32, 16)

        def t_of(i):
            return i * nw + wid

        def valid(i):
            return t_of(i) < n_reg

        def load(i, p, start):
            cp = pltpu.make_async_copy(
                wt_hbm.at[:, pl.ds(t_of(i) * CH, CH)], blk[p], lsem[p])
            cp.start() if start else cp.wait()

        def transpose(src, dst, nu):
            ncol = 2 * nu

            def sbody(s, carry):
                rot = jnp.bitwise_and(iota + s, 15)
                work = []
                for c0 in range(0, ncol, 16):
                    cvec = iota + c0
                    u_row = lax.shift_right_logical(cvec, 1)
                    j_base = lax.shift_left(jnp.bitwise_and(cvec, 1), 6)
                    for e0 in range(0, emb, 16):
                        erow = rot + e0
                        tv = plsc.load_gather(src, [erow, cvec])
                        work.append((u_row, j_base + erow, tv))
                for u_row, jcol, tv in work:
                    plsc.store_scatter(dst, [u_row, jcol], tv)
                return carry

            lax.fori_loop(0, 16, sbody, 0)

        def store(i, p, start):
            cp = pltpu.make_async_copy(
                tblk[p], w2_hbm.at[pl.ds(t_of(i) * emb, emb), :], ssem[p])
            cp.start() if start else cp.wait()

        for p in range(_NS):
            @pl.when(valid(p))
            def _(p=p):
                load(p, p, True)

        def body(j, carry):
            for p in range(_NS):
                i = _NS * j + p

                @pl.when(valid(i))
                def _(i=i, p=p):
                    load(i, p, False)

                @pl.when((j >= 1) & valid(i - _NS))
                def _(i=i, p=p):
                    store(i - _NS, p, False)

                @pl.when(valid(i))
                def _(i=i, p=p):
                    transpose(blk[p], tblk[p], emb)
                    store(i, p, True)

                @pl.when(valid(i + _NS))
                def _(i=i, p=p):
                    load(i + _NS, p, True)
            return carry

        lax.fori_loop(0, n_iter, body, 0)

        for p in range(_NS):
            i = _NS * (n_iter - 1) + p

            @pl.when(valid(i))
            def _(i=i, p=p):
                store(i, p, False)

        if tail:
            @pl.when(wid == 0)
            def _():
                pltpu.sync_copy(wt_hbm.at[:, pl.ds(n_reg * CH, tail)], tb)
                transpose(tb, ttb, tail // 2)
                pltpu.sync_copy(
                    ttb, w2_hbm.at[pl.ds(n_reg * emb, tail // 2), :])

    return repack_kernel


def _make_lookup(vocab: int, emb: int, b_dim: int, l_dim: int):
    info = plsc.get_sparse_core_info()
    nc, ns = info.num_cores, info.num_subcores
    nw = nc * ns
    CH = 128
    n_bc = b_dim // CH
    n_chunks = l_dim * n_bc
    per_w = n_chunks // nw
    assert n_chunks % (nw * _NS) == 0
    n_iter = per_w // _NS

    mesh = plsc.VectorSubcoreMesh(core_axis_name="c", subcore_axis_name="s")

    @functools.partial(
        pl.kernel,
        mesh=mesh,
        out_type=jax.ShapeDtypeStruct((l_dim, emb, b_dim), jnp.float32),
        scratch_types=(
            [pltpu.VMEM((CH,), jnp.int32) for _ in range(3 * _NS)]
            + [pltpu.VMEM((CH, 2 * emb), jnp.float32) for _ in range(_NS)]
            + [pltpu.VMEM((emb, CH), jnp.float32) for _ in range(_NS)]
            + [pltpu.SemaphoreType.DMA for _ in range(3 * _NS)]
        ),
        compiler_params=_CP,
    )
    def lookup_kernel(mt_hbm, w2_hbm, out_hbm, *scr):
        idxraw = scr[0:_NS]
        gidx = scr[_NS:2 * _NS]
        par = scr[2 * _NS:3 * _NS]
        rows = scr[3 * _NS:4 * _NS]
        trows = scr[4 * _NS:5 * _NS]
        isem = scr[5 * _NS:6 * _NS]
        gsem = scr[6 * _NS:7 * _NS]
        osem = scr[7 * _NS:8 * _NS]
        wid = lax.axis_index("s") * nc + lax.axis_index("c")
        base = wid * per_w
        iota = lax.iota(jnp.int32, 16)

        def lc(t):
            return lax.div(t, n_bc), lax.rem(t, n_bc) * CH

        def idx_load(t, p, start):
            l, c = lc(base + t)
            cp = pltpu.make_async_copy(
                mt_hbm.at[pl.ds(l * b_dim + c, CH)], idxraw[p], isem[p])
            cp.start() if start else cp.wait()

        def process(p):
            for k in range(0, CH, 16):
                v = idxraw[p][pl.ds(k, 16)]
                gidx[p][pl.ds(k, 16)] = lax.shift_right_logical(v, 1)
                par[p][pl.ds(k, 16)] = lax.shift_left(
                    jnp.bitwise_and(v, 1), 6)

        def gather(p, start):
            cp = pltpu.make_async_copy(w2_hbm.at[gidx[p]], rows[p], gsem[p])
            cp.start() if start else cp.wait()

        def transpose(p):
            def sbody(s, carry):
                rot = jnp.bitwise_and(iota + s, 15)
                work = []
                for k in range(0, CH, 16):
                    rowk = iota + k
                    pk = par[p][pl.ds(k, 16)]
                    for e0 in range(0, emb, 16):
                        erow = rot + e0
                        tv = plsc.load_gather(rows[p], [rowk, pk + erow])
                        work.append((erow, rowk, tv))
                for erow, rowk, tv in work:
                    plsc.store_scatter(trows[p], [erow, rowk], tv)
                return carry

            lax.fori_loop(0, 16, sbody, 0)

        def store(t, p, start):
            l, c = lc(base + t)
            cp = pltpu.make_async_copy(
                trows[p], out_hbm.at[l, :, pl.ds(c, CH)], osem[p])
            cp.start() if start else cp.wait()

        for p in range(_NS):
            idx_load(p, p, True)
        for p in range(_NS - 1):
            idx_load(p, p, False)
            process(p)
            gather(p, True)

        def body(j, carry):
            for p in range(_NS):
                t = _NS * j + p
                q = (p + _NS - 1) % _NS

                gather(p, False)

                @pl.when(t + _NS - 1 < per_w)
                def _(t=t, q=q):
                    idx_load(t + _NS - 1, q, False)
                    process(q)
                    gather(q, True)

                @pl.when(t + _NS < per_w)
                def _(t=t, p=p):
                    idx_load(t + _NS, p, True)

                @pl.when(j >= 1)
                def _(p=p):
                    store(_NS * j + p - _NS, p, False)

                transpose(p)
                store(t, p, True)
            return carry

        lax.fori_loop(0, n_iter, body, 0)

        for p in range(_NS):
            store(per_w - _NS + p, p, False)

    return lookup_kernel


def kernel(mask, weight):
    b, l = mask.shape
    vocab, emb = weight.shape
    mtf = mask.T.reshape(-1)
    wt = weight.T
    w2 = _make_repack(vocab, emb)(wt)
    out = _make_lookup(vocab, emb, b, l)(mtf, w2)
    return out.transpose(2, 0, 1)
